# Initial kernel scaffold; baseline (speedup 1.0000x reference)
import jax
import jax.numpy as jnp
from jax.experimental import pallas as pl


def kernel(x, rel_repr, edge_index, edge_type, edge_norm, in_w, out_w, loop_w, w_rel, loop_rel, bias, bn_gamma, bn_beta):
    raise NotImplementedError("write your pallas kernel here")


# speedup vs baseline: 3.2211x; 3.2211x over previous
import functools

import jax
import jax.numpy as jnp
from jax import lax
from jax.experimental import pallas as pl
from jax.experimental.pallas import tpu as pltpu
from jax.experimental.pallas import tpu_sc as plsc

N_NODES = 10000
D = 128
NREL = 200
NT = 2 * NREL

NC, NS = 2, 16
NW = NC * NS
NODES_PER_W = (N_NODES + NW - 1) // NW
ROWS_PER_W = NODES_PER_W * NT
S_ROWS = NW * NODES_PER_W

CHUNK = 1280


def _tc_prep(rel_ref, in_w_ref, out_w_ref, w_rel_ref, circ_ref, loop_w_ref,
             dst_ref, typ_ref, t_ref, weff_ref, out2_ref, flat_ref, *, half):
    rel = rel_ref[...]
    ti = jax.nn.softmax(jnp.dot(rel, in_w_ref[...],
                                preferred_element_type=jnp.float32), axis=-1)
    to = jax.nn.softmax(jnp.dot(rel, out_w_ref[...],
                                preferred_element_type=jnp.float32), axis=-1)
    t_ref[...] = jnp.concatenate([ti, to], axis=0)
    weff_ref[...] = jnp.dot(circ_ref[...], loop_w_ref[...],
                            preferred_element_type=jnp.float32)
    out2_ref[...] = jnp.dot(rel, w_rel_ref[...],
                            preferred_element_type=jnp.float32)
    shp = dst_ref.shape
    e = (lax.broadcasted_iota(jnp.int32, shp, 0) * shp[1]
         + lax.broadcasted_iota(jnp.int32, shp, 1))
    side = jnp.where(e >= half, jnp.int32(NREL), jnp.int32(0))
    flat_ref[...] = dst_ref[...] * NT + typ_ref[...] + side


def _sc_hist(flat_hbm, norm_hbm, s_hbm, s_v, fbuf, nbuf, *, n_edges):
    wid = lax.axis_index("s") * NC + lax.axis_index("c")
    base = wid * ROWS_PER_W

    zero = jnp.zeros((16,), jnp.float32)

    def zbody(i, carry):
        s_v[pl.ds(i * 16, 16)] = zero
        return carry

    lax.fori_loop(0, ROWS_PER_W // 16, zbody, 0, unroll=8)

    def chunk_body(g, carry):
        pltpu.sync_copy(flat_hbm.at[pl.ds(g * CHUNK, CHUNK)], fbuf)
        pltpu.sync_copy(norm_hbm.at[pl.ds(g * CHUNK, CHUNK)], nbuf)

        def vbody(j, c2):
            fl = fbuf[pl.ds(j * 16, 16)]
            nm = nbuf[pl.ds(j * 16, 16)]
            loc = fl - base
            msk = (loc >= 0) & (loc < ROWS_PER_W)
            locc = jnp.minimum(jnp.maximum(loc, 0), ROWS_PER_W - 1)
            plsc.addupdate_scatter(s_v, [locc], nm, mask=msk)
            return c2

        lax.fori_loop(0, CHUNK // 16, vbody, 0, unroll=4)
        return carry

    lax.fori_loop(0, n_edges // CHUNK, chunk_body, 0)
    pltpu.sync_copy(s_v, s_hbm.at[pl.ds(base, ROWS_PER_W)])


NB = 10
BR = N_NODES // NB


def _tc_agg(s_ref, x_ref, t_ref, weff_ref, bias_ref, h_ref, stats_ref, acc):
    i = pl.program_id(0)
    hb = jnp.dot(s_ref[...], t_ref[...],
                 preferred_element_type=jnp.float32) / 3.0
    hb = hb + jnp.dot(x_ref[...], weff_ref[...],
                      preferred_element_type=jnp.float32) / 3.0
    hb = hb + bias_ref[...]
    h_ref[...] = hb

    @pl.when(i == 0)
    def _():
        acc[...] = jnp.zeros_like(acc)

    acc[0:1, :] += jnp.sum(hb, axis=0, keepdims=True)
    acc[1:2, :] += jnp.sum(hb * hb, axis=0, keepdims=True)
    stats_ref[...] = acc[...]


def _tc_bn(h_ref, stats_ref, gamma_ref, beta_ref, o_ref):
    s = stats_ref[...]
    n = jnp.float32(N_NODES)
    mean = s[0:1, :] / n
    var = s[1:2, :] / n - mean * mean
    o_ref[...] = ((h_ref[...] - mean) / jnp.sqrt(var + 1e-5)
                  * gamma_ref[...] + beta_ref[...])


def kernel(x, rel_repr, edge_index, edge_type, edge_norm, in_w, out_w,
           loop_w, w_rel, loop_rel, bias, bn_gamma, bn_beta):
    n_edges = edge_type.shape[0]
    half = n_edges // 2
    dst2 = edge_index[1].reshape(n_edges // D, D)
    typ2 = edge_type.reshape(n_edges // D, D)

    r = loop_rel[0]
    idx = (jnp.arange(D)[:, None] + jnp.arange(D)[None, :]) % D
    circ = r[idx]

    t_tab, weff, out2, flat2 = pl.pallas_call(
        functools.partial(_tc_prep, half=half),
        out_shape=[
            jax.ShapeDtypeStruct((NT, D), jnp.float32),
            jax.ShapeDtypeStruct((D, D), jnp.float32),
            jax.ShapeDtypeStruct((NREL, D), jnp.float32),
            jax.ShapeDtypeStruct((n_edges // D, D), jnp.int32),
        ],
    )(rel_repr, in_w, out_w, w_rel, circ, loop_w, dst2, typ2)

    flat = flat2.reshape(n_edges)

    mesh = plsc.VectorSubcoreMesh(core_axis_name="c", subcore_axis_name="s",
                                  num_cores=NC, num_subcores=NS)
    s_flat = pl.kernel(
        functools.partial(_sc_hist, n_edges=n_edges),
        out_type=jax.ShapeDtypeStruct((S_ROWS * NT,), jnp.float32),
        mesh=mesh,
        compiler_params=pltpu.CompilerParams(needs_layout_passes=False),
        scratch_types=[
            pltpu.VMEM((ROWS_PER_W,), jnp.float32),
            pltpu.VMEM((CHUNK,), jnp.int32),
            pltpu.VMEM((CHUNK,), jnp.float32),
        ],
    )(flat, edge_norm)
    s_mat = s_flat.reshape(S_ROWS, NT)

    h_raw, stats = pl.pallas_call(
        _tc_agg,
        grid=(NB,),
        in_specs=[
            pl.BlockSpec((BR, NT), lambda i: (i, 0)),
            pl.BlockSpec((BR, D), lambda i: (i, 0)),
            pl.BlockSpec((NT, D), lambda i: (0, 0)),
            pl.BlockSpec((D, D), lambda i: (0, 0)),
            pl.BlockSpec((1, D), lambda i: (0, 0)),
        ],
        out_specs=[
            pl.BlockSpec((BR, D), lambda i: (i, 0)),
            pl.BlockSpec((8, D), lambda i: (0, 0)),
        ],
        out_shape=[
            jax.ShapeDtypeStruct((N_NODES, D), jnp.float32),
            jax.ShapeDtypeStruct((8, D), jnp.float32),
        ],
        scratch_shapes=[pltpu.VMEM((8, D), jnp.float32)],
    )(s_mat, x, t_tab, weff, bias.reshape(1, D))

    h = pl.pallas_call(
        _tc_bn,
        grid=(NB,),
        in_specs=[
            pl.BlockSpec((BR, D), lambda i: (i, 0)),
            pl.BlockSpec((8, D), lambda i: (0, 0)),
            pl.BlockSpec((1, D), lambda i: (0, 0)),
            pl.BlockSpec((1, D), lambda i: (0, 0)),
        ],
        out_specs=pl.BlockSpec((BR, D), lambda i: (i, 0)),
        out_shape=jax.ShapeDtypeStruct((N_NODES, D), jnp.float32),
    )(h_raw, stats, bn_gamma.reshape(1, D), bn_beta.reshape(1, D))

    return (h, out2)


# speedup vs baseline: 4.5480x; 1.4119x over previous
import functools

import jax
import jax.numpy as jnp
from jax import lax
from jax.experimental import pallas as pl
from jax.experimental.pallas import tpu as pltpu
from jax.experimental.pallas import tpu_sc as plsc

N_NODES = 10000
D = 128
NREL = 200

NC, NS = 2, 16
NPASS = 2
PASS_WORDS = N_NODES * NREL // NPASS
PASS_ROWS = PASS_WORDS // 16
ROWS_PER_TILE = 3912
PASS_ROWS_PAD = ROWS_PER_TILE * NS
DUMP_ROW = PASS_ROWS

CH = 2000
BATCH = 125
NBATCH = CH // BATCH


def _tc_prep(rel_ref, in_w_ref, out_w_ref, w_rel_ref, circ_ref, loop_w_ref,
             dst_ref, typ_ref, tin_ref, tout_ref, weff_ref, out2_ref,
             flat_ref):
    rel = rel_ref[...]
    tin_ref[...] = jax.nn.softmax(
        jnp.dot(rel, in_w_ref[...], preferred_element_type=jnp.float32),
        axis=-1)
    tout_ref[...] = jax.nn.softmax(
        jnp.dot(rel, out_w_ref[...], preferred_element_type=jnp.float32),
        axis=-1)
    weff_ref[...] = jnp.dot(circ_ref[...], loop_w_ref[...],
                            preferred_element_type=jnp.float32)
    out2_ref[...] = jnp.dot(rel, w_rel_ref[...],
                            preferred_element_type=jnp.float32)
    flat_ref[...] = dst_ref[...] * NREL + typ_ref[...]


def _sc_hist(flat_hbm, norm_hbm, s_hbm, shared, onehot, flatbuf, normbuf,
             rowbuf, sem, *, half):
    c = lax.axis_index("c")
    s = lax.axis_index("s")
    edges_per_tile = half // NS
    nch = edges_per_tile // CH
    base_e = c * half + s * edges_per_tile
    row0 = s * ROWS_PER_TILE

    zero16 = jnp.zeros((16,), jnp.float32)
    iota16 = lax.iota(jnp.int32, 16)

    def zb(i, carry):
        onehot[i, :] = zero16
        return carry

    lax.fori_loop(0, CH, zb, 0, unroll=8)

    def clean(j, carry):
        fl = flatbuf[pl.ds(j * 16, 16)]
        col = lax.bitwise_and(fl, 15)
        rowi = j * 16 + iota16
        plsc.store_scatter(onehot, [rowi, col], zero16)
        return carry

    for p in range(NPASS):
        for off, sz in ((0, 2000), (2000, 1912)):
            pltpu.sync_copy(onehot.at[pl.ds(0, sz)],
                            shared.at[pl.ds(row0 + off, sz)])
        plsc.subcore_barrier()

        wbase = p * PASS_WORDS

        def build(j, carry):
            fl = flatbuf[pl.ds(j * 16, 16)]
            nm = normbuf[pl.ds(j * 16, 16)]
            loc = fl - wbase
            inr = (loc >= 0) & (loc < PASS_WORDS)
            col = lax.bitwise_and(loc, 15)
            row = lax.shift_right_logical(loc, 4)
            rowe = jnp.where(inr, row, DUMP_ROW)
            nme = jnp.where(inr, nm, 0.0)
            rowi = j * 16 + iota16
            plsc.store_scatter(onehot, [rowi, col], nme)
            rb_r = lax.div(rowi, BATCH)
            rb_c = lax.rem(rowi, BATCH)
            plsc.store_scatter(rowbuf, [rb_r, rb_c], rowe)
            return carry

        for g in range(nch):
            eb = base_e + g * CH
            pltpu.sync_copy(flat_hbm.at[pl.ds(eb, CH)], flatbuf)
            pltpu.sync_copy(norm_hbm.at[pl.ds(eb, CH)], normbuf)
            lax.fori_loop(0, CH // 16, build, 0, unroll=4)
            cps = [
                pltpu.async_copy(onehot.at[pl.ds(b * BATCH, BATCH)],
                                 shared.at[rowbuf.at[b]], sem, add=True)
                for b in range(NBATCH)
            ]
            for cp in cps:
                cp.wait()
            lax.fori_loop(0, CH // 16, clean, 0, unroll=4)

        plsc.subcore_barrier()
        for off, sz in ((0, 2000), (2000, 1912)):
            pltpu.sync_copy(shared.at[pl.ds(row0 + off, sz)],
                            s_hbm.at[c, p, pl.ds(row0 + off, sz)])
        plsc.subcore_barrier()


NB = 10
BR = N_NODES // NB


def _tc_agg(s0_ref, s1_ref, x_ref, tin_ref, tout_ref, weff_ref, bias_ref,
            h_ref, stats_ref, acc):
    i = pl.program_id(0)
    hb = jnp.dot(s0_ref[...], tin_ref[...],
                 preferred_element_type=jnp.float32)
    hb = hb + jnp.dot(s1_ref[...], tout_ref[...],
                      preferred_element_type=jnp.float32)
    hb = hb / 3.0
    hb = hb + jnp.dot(x_ref[...], weff_ref[...],
                      preferred_element_type=jnp.float32) / 3.0
    hb = hb + bias_ref[...]
    h_ref[...] = hb

    @pl.when(i == 0)
    def _():
        acc[...] = jnp.zeros_like(acc)

    acc[0:1, :] += jnp.sum(hb, axis=0, keepdims=True)
    acc[1:2, :] += jnp.sum(hb * hb, axis=0, keepdims=True)
    stats_ref[...] = acc[...]


def _tc_bn(h_ref, stats_ref, gamma_ref, beta_ref, o_ref):
    st = stats_ref[...]
    n = jnp.float32(N_NODES)
    mean = st[0:1, :] / n
    var = st[1:2, :] / n - mean * mean
    o_ref[...] = ((h_ref[...] - mean) / jnp.sqrt(var + 1e-5)
                  * gamma_ref[...] + beta_ref[...])


def kernel(x, rel_repr, edge_index, edge_type, edge_norm, in_w, out_w,
           loop_w, w_rel, loop_rel, bias, bn_gamma, bn_beta):
    n_edges = edge_type.shape[0]
    half = n_edges // 2
    dst2 = edge_index[1].reshape(n_edges // D, D)
    typ2 = edge_type.reshape(n_edges // D, D)

    r = loop_rel[0]
    idx = (jnp.arange(D)[:, None] + jnp.arange(D)[None, :]) % D
    circ = r[idx]

    t_in, t_out, weff, out2, flat2 = pl.pallas_call(
        _tc_prep,
        out_shape=[
            jax.ShapeDtypeStruct((NREL, D), jnp.float32),
            jax.ShapeDtypeStruct((NREL, D), jnp.float32),
            jax.ShapeDtypeStruct((D, D), jnp.float32),
            jax.ShapeDtypeStruct((NREL, D), jnp.float32),
            jax.ShapeDtypeStruct((n_edges // D, D), jnp.int32),
        ],
    )(rel_repr, in_w, out_w, w_rel, circ, loop_w, dst2, typ2)

    flat = flat2.reshape(n_edges)

    mesh = plsc.VectorSubcoreMesh(core_axis_name="c", subcore_axis_name="s",
                                  num_cores=NC, num_subcores=NS)
    s_out = pl.kernel(
        functools.partial(_sc_hist, half=half),
        out_type=jax.ShapeDtypeStruct((NC, NPASS, PASS_ROWS_PAD, 16),
                                      jnp.float32),
        mesh=mesh,
        compiler_params=pltpu.CompilerParams(needs_layout_passes=False,
                                             use_tc_tiling_on_sc=False),
        scratch_types=[
            pltpu.VMEM_SHARED((PASS_ROWS_PAD, 16), jnp.float32),
            pltpu.VMEM((CH, 16), jnp.float32),
            pltpu.VMEM((CH,), jnp.int32),
            pltpu.VMEM((CH,), jnp.float32),
            pltpu.VMEM((NBATCH, BATCH), jnp.int32),
            pltpu.SemaphoreType.DMA,
        ],
    )(flat, edge_norm)

    s_lin = s_out.reshape(NC, NPASS, PASS_ROWS_PAD * 16)
    s0 = s_lin[0, :, :PASS_WORDS].reshape(N_NODES, NREL)
    s1 = s_lin[1, :, :PASS_WORDS].reshape(N_NODES, NREL)

    h_raw, stats = pl.pallas_call(
        _tc_agg,
        grid=(NB,),
        in_specs=[
            pl.BlockSpec((BR, NREL), lambda i: (i, 0)),
            pl.BlockSpec((BR, NREL), lambda i: (i, 0)),
            pl.BlockSpec((BR, D), lambda i: (i, 0)),
            pl.BlockSpec((NREL, D), lambda i: (0, 0)),
            pl.BlockSpec((NREL, D), lambda i: (0, 0)),
            pl.BlockSpec((D, D), lambda i: (0, 0)),
            pl.BlockSpec((1, D), lambda i: (0, 0)),
        ],
        out_specs=[
            pl.BlockSpec((BR, D), lambda i: (i, 0)),
            pl.BlockSpec((8, D), lambda i: (0, 0)),
        ],
        out_shape=[
            jax.ShapeDtypeStruct((N_NODES, D), jnp.float32),
            jax.ShapeDtypeStruct((8, D), jnp.float32),
        ],
        scratch_shapes=[pltpu.VMEM((8, D), jnp.float32)],
    )(s0, s1, x, t_in, t_out, weff, bias.reshape(1, D))

    h = pl.pallas_call(
        _tc_bn,
        grid=(NB,),
        in_specs=[
            pl.BlockSpec((BR, D), lambda i: (i, 0)),
            pl.BlockSpec((8, D), lambda i: (0, 0)),
            pl.BlockSpec((1, D), lambda i: (0, 0)),
            pl.BlockSpec((1, D), lambda i: (0, 0)),
        ],
        out_specs=pl.BlockSpec((BR, D), lambda i: (i, 0)),
        out_shape=jax.ShapeDtypeStruct((N_NODES, D), jnp.float32),
    )(h_raw, stats, bn_gamma.reshape(1, D), bn_beta.reshape(1, D))

    return (h, out2)


# speedup vs baseline: 4.8168x; 1.0591x over previous
import functools

import jax
import jax.numpy as jnp
from jax import lax
from jax.experimental import pallas as pl
from jax.experimental.pallas import tpu as pltpu
from jax.experimental.pallas import tpu_sc as plsc

N_NODES = 10000
D = 128
NREL = 200

NC, NS = 2, 16
NPASS = 2
PASS_WORDS = N_NODES * NREL // NPASS
PASS_ROWS = PASS_WORDS // 16
ZERO_RPT = 3907
SHARED_ROWS = ZERO_RPT * NS
DUMP_ROW = PASS_ROWS
OUT_RPT = 3906

CH = 2000
BATCH = 125
NBATCH = CH // BATCH


def _tc_prep(rel_ref, in_w_ref, out_w_ref, w_rel_ref, circ_ref, loop_w_ref,
             dst_ref, typ_ref, tin_ref, tout_ref, weff_ref, out2_ref,
             flat_ref):
    rel = rel_ref[...]
    tin_ref[...] = jax.nn.softmax(
        jnp.dot(rel, in_w_ref[...], preferred_element_type=jnp.float32),
        axis=-1)
    tout_ref[...] = jax.nn.softmax(
        jnp.dot(rel, out_w_ref[...], preferred_element_type=jnp.float32),
        axis=-1)
    weff_ref[...] = jnp.dot(circ_ref[...], loop_w_ref[...],
                            preferred_element_type=jnp.float32)
    out2_ref[...] = jnp.dot(rel, w_rel_ref[...],
                            preferred_element_type=jnp.float32)
    flat_ref[...] = dst_ref[...] * NREL + typ_ref[...]


def _sc_hist(flat_hbm, norm_hbm, s_hbm, shared, onehot, flatbuf, normbuf,
             rowbuf, sem, *, half):
    c = lax.axis_index("c")
    s = lax.axis_index("s")
    edges_per_tile = half // NS
    nch = edges_per_tile // CH
    base_e = c * half + s * edges_per_tile
    zrow0 = s * ZERO_RPT
    orow0 = s * OUT_RPT

    zero16 = jnp.zeros((16,), jnp.float32)
    iota16 = lax.iota(jnp.int32, 16)

    def zb(i, carry):
        onehot[i, :] = zero16
        return carry

    lax.fori_loop(0, CH, zb, 0, unroll=8)

    def clean(j, carry):
        fl = flatbuf[pl.ds(j * 16, 16)]
        col = lax.bitwise_and(fl, 15)
        rowi = j * 16 + iota16
        plsc.store_scatter(onehot, [rowi, col], zero16)
        return carry

    for p in range(NPASS):
        for off, sz in ((0, 2000), (2000, 1907)):
            pltpu.sync_copy(onehot.at[pl.ds(0, sz)],
                            shared.at[pl.ds(zrow0 + off, sz)])
        plsc.subcore_barrier()

        wbase = p * PASS_WORDS

        def build(j, carry):
            fl = flatbuf[pl.ds(j * 16, 16)]
            nm = normbuf[pl.ds(j * 16, 16)]
            loc = fl - wbase
            inr = (loc >= 0) & (loc < PASS_WORDS)
            col = lax.bitwise_and(loc, 15)
            row = lax.shift_right_logical(loc, 4)
            rowe = jnp.where(inr, row, DUMP_ROW)
            nme = jnp.where(inr, nm, 0.0)
            rowi = j * 16 + iota16
            plsc.store_scatter(onehot, [rowi, col], nme)
            rb_r = lax.div(rowi, BATCH)
            rb_c = lax.rem(rowi, BATCH)
            plsc.store_scatter(rowbuf, [rb_r, rb_c], rowe)
            return carry

        for g in range(nch):
            eb = base_e + g * CH
            pltpu.sync_copy(flat_hbm.at[pl.ds(eb, CH)], flatbuf)
            pltpu.sync_copy(norm_hbm.at[pl.ds(eb, CH)], normbuf)
            lax.fori_loop(0, CH // 16, build, 0, unroll=4)
            cps = [
                pltpu.async_copy(onehot.at[pl.ds(b * BATCH, BATCH)],
                                 shared.at[rowbuf.at[b]], sem, add=True)
                for b in range(NBATCH)
            ]
            for cp in cps:
                cp.wait()
            lax.fori_loop(0, CH // 16, clean, 0, unroll=4)

        plsc.subcore_barrier()
        for off, sz in ((0, 2000), (2000, 1906)):
            pltpu.sync_copy(shared.at[pl.ds(orow0 + off, sz)],
                            s_hbm.at[c, p, pl.ds(orow0 + off, sz)])

        @pl.when(s == 0)
        def _copy_tail():
            pltpu.sync_copy(shared.at[pl.ds(OUT_RPT * NS, 4)],
                            s_hbm.at[c, p, pl.ds(OUT_RPT * NS, 4)])

        plsc.subcore_barrier()


NB = 10
BR = N_NODES // NB


def _tc_agg(s0_ref, s1_ref, x_ref, tin_ref, tout_ref, weff_ref, bias_ref,
            h_ref, stats_ref, acc):
    i = pl.program_id(0)
    hb = jnp.dot(s0_ref[...], tin_ref[...],
                 preferred_element_type=jnp.float32)
    hb = hb + jnp.dot(s1_ref[...], tout_ref[...],
                      preferred_element_type=jnp.float32)
    hb = hb / 3.0
    hb = hb + jnp.dot(x_ref[...], weff_ref[...],
                      preferred_element_type=jnp.float32) / 3.0
    hb = hb + bias_ref[...]
    h_ref[...] = hb

    @pl.when(i == 0)
    def _():
        acc[...] = jnp.zeros_like(acc)

    acc[0:1, :] += jnp.sum(hb, axis=0, keepdims=True)
    acc[1:2, :] += jnp.sum(hb * hb, axis=0, keepdims=True)
    stats_ref[...] = acc[...]


def _tc_bn(h_ref, stats_ref, gamma_ref, beta_ref, o_ref):
    st = stats_ref[...]
    n = jnp.float32(N_NODES)
    mean = st[0:1, :] / n
    var = st[1:2, :] / n - mean * mean
    o_ref[...] = ((h_ref[...] - mean) / jnp.sqrt(var + 1e-5)
                  * gamma_ref[...] + beta_ref[...])


def kernel(x, rel_repr, edge_index, edge_type, edge_norm, in_w, out_w,
           loop_w, w_rel, loop_rel, bias, bn_gamma, bn_beta):
    n_edges = edge_type.shape[0]
    half = n_edges // 2
    dst2 = edge_index[1].reshape(n_edges // D, D)
    typ2 = edge_type.reshape(n_edges // D, D)

    r = loop_rel[0]
    rr = jnp.concatenate([r, r])
    circ = jnp.stack([lax.slice(rr, (j,), (j + D,)) for j in range(D)])

    t_in, t_out, weff, out2, flat2 = pl.pallas_call(
        _tc_prep,
        out_shape=[
            jax.ShapeDtypeStruct((NREL, D), jnp.float32),
            jax.ShapeDtypeStruct((NREL, D), jnp.float32),
            jax.ShapeDtypeStruct((D, D), jnp.float32),
            jax.ShapeDtypeStruct((NREL, D), jnp.float32),
            jax.ShapeDtypeStruct((n_edges // D, D), jnp.int32),
        ],
    )(rel_repr, in_w, out_w, w_rel, circ, loop_w, dst2, typ2)

    flat = flat2.reshape(n_edges)

    mesh = plsc.VectorSubcoreMesh(core_axis_name="c", subcore_axis_name="s",
                                  num_cores=NC, num_subcores=NS)
    s_out = pl.kernel(
        functools.partial(_sc_hist, half=half),
        out_type=jax.ShapeDtypeStruct((NC, NPASS, PASS_ROWS, 16),
                                      jnp.float32),
        mesh=mesh,
        compiler_params=pltpu.CompilerParams(needs_layout_passes=False,
                                             use_tc_tiling_on_sc=False),
        scratch_types=[
            pltpu.VMEM_SHARED((SHARED_ROWS, 16), jnp.float32),
            pltpu.VMEM((CH, 16), jnp.float32),
            pltpu.VMEM((CH,), jnp.int32),
            pltpu.VMEM((CH,), jnp.float32),
            pltpu.VMEM((NBATCH, BATCH), jnp.int32),
            pltpu.SemaphoreType.DMA,
        ],
    )(flat, edge_norm)

    s_lin = s_out.reshape(NC, N_NODES, NREL)
    s0 = s_lin[0]
    s1 = s_lin[1]

    h_raw, stats = pl.pallas_call(
        _tc_agg,
        grid=(NB,),
        in_specs=[
            pl.BlockSpec((BR, NREL), lambda i: (i, 0)),
            pl.BlockSpec((BR, NREL), lambda i: (i, 0)),
            pl.BlockSpec((BR, D), lambda i: (i, 0)),
            pl.BlockSpec((NREL, D), lambda i: (0, 0)),
            pl.BlockSpec((NREL, D), lambda i: (0, 0)),
            pl.BlockSpec((D, D), lambda i: (0, 0)),
            pl.BlockSpec((1, D), lambda i: (0, 0)),
        ],
        out_specs=[
            pl.BlockSpec((BR, D), lambda i: (i, 0)),
            pl.BlockSpec((8, D), lambda i: (0, 0)),
        ],
        out_shape=[
            jax.ShapeDtypeStruct((N_NODES, D), jnp.float32),
            jax.ShapeDtypeStruct((8, D), jnp.float32),
        ],
        scratch_shapes=[pltpu.VMEM((8, D), jnp.float32)],
    )(s0, s1, x, t_in, t_out, weff, bias.reshape(1, D))

    h = pl.pallas_call(
        _tc_bn,
        grid=(NB,),
        in_specs=[
            pl.BlockSpec((BR, D), lambda i: (i, 0)),
            pl.BlockSpec((8, D), lambda i: (0, 0)),
            pl.BlockSpec((1, D), lambda i: (0, 0)),
            pl.BlockSpec((1, D), lambda i: (0, 0)),
        ],
        out_specs=pl.BlockSpec((BR, D), lambda i: (i, 0)),
        out_shape=jax.ShapeDtypeStruct((N_NODES, D), jnp.float32),
    )(h_raw, stats, bn_gamma.reshape(1, D), bn_beta.reshape(1, D))

    return (h, out2)


# speedup vs baseline: 8.2299x; 1.7086x over previous
import functools

import jax
import jax.numpy as jnp
from jax import lax
from jax.experimental import pallas as pl
from jax.experimental.pallas import tpu as pltpu
from jax.experimental.pallas import tpu_sc as plsc

N_NODES = 10000
D = 128
NREL = 200
DSTP = 10112

NC, NS = 2, 16
SC_WORDS = NREL * DSTP
SHARED_ROWS = SC_WORDS // 16
ROWS_PER_TILE = SHARED_ROWS // NS

CH = 400
BATCH = 80
NBATCH = CH // BATCH


def _tc_prep(rel_ref, in_w_ref, out_w_ref, w_rel_ref, circ_ref, loop_w_ref,
             dst_ref, typ_ref, tin_ref, tout_ref, weff_ref, out2_ref,
             flat_ref):
    rel = rel_ref[...]
    tin_ref[...] = jax.nn.softmax(
        jnp.dot(rel, in_w_ref[...], preferred_element_type=jnp.float32),
        axis=-1)
    tout_ref[...] = jax.nn.softmax(
        jnp.dot(rel, out_w_ref[...], preferred_element_type=jnp.float32),
        axis=-1)
    weff_ref[...] = jnp.dot(circ_ref[...], loop_w_ref[...],
                            preferred_element_type=jnp.float32)
    out2_ref[...] = jnp.dot(rel, w_rel_ref[...],
                            preferred_element_type=jnp.float32)
    flat_ref[...] = typ_ref[...] * DSTP + dst_ref[...]


def _sc_hist(flat_hbm, norm_hbm, zero_hbm, s_hbm, shared, oh0, oh1, flatbuf,
             normbuf, colb0, colb1, rowb0, rowb1, sem0, sem1, *, half):
    c = lax.axis_index("c")
    s = lax.axis_index("s")
    edges_per_tile = half // NS
    nch = edges_per_tile // CH
    base_e = c * half + s * edges_per_tile
    zrow0 = s * ROWS_PER_TILE

    zero16 = jnp.zeros((16,), jnp.float32)
    iota16 = lax.iota(jnp.int32, 16)

    ohs = (oh0, oh1)
    colbs = (colb0, colb1)
    rowbs = (rowb0, rowb1)
    sems = (sem0, sem1)

    def zb(i, carry):
        oh0[i, :] = zero16
        oh1[i, :] = zero16
        return carry

    lax.fori_loop(0, BATCH, zb, 0, unroll=8)
    pltpu.sync_copy(zero_hbm, shared.at[pl.ds(zrow0, ROWS_PER_TILE)])
    plsc.subcore_barrier()

    def chunk_body(g, carry):
        eb = base_e + g * CH
        pltpu.sync_copy(flat_hbm.at[pl.ds(eb, CH)], flatbuf)
        pltpu.sync_copy(norm_hbm.at[pl.ds(eb, CH)], normbuf)

        cps = [None] * NBATCH
        for b in range(NBATCH):
            sl = b % 2
            oh, colb, rowb = ohs[sl], colbs[sl], rowbs[sl]
            if b >= 2:
                cps[b - 2].wait()

                def clean(i, carry2):
                    col = colb[pl.ds(i * 16, 16)]
                    rowi = i * 16 + iota16
                    plsc.store_scatter(oh, [rowi, col], zero16)
                    return carry2

                lax.fori_loop(0, BATCH // 16, clean, 0, unroll=5)

            def build(i, carry2):
                fl = flatbuf[pl.ds(b * BATCH + i * 16, 16)]
                nm = normbuf[pl.ds(b * BATCH + i * 16, 16)]
                col = lax.bitwise_and(fl, 15)
                row = lax.shift_right_logical(fl, 4)
                rowi = i * 16 + iota16
                plsc.store_scatter(oh, [rowi, col], nm)
                colb[pl.ds(i * 16, 16)] = col
                rowb[pl.ds(i * 16, 16)] = row
                return carry2

            lax.fori_loop(0, BATCH // 16, build, 0, unroll=5)
            cps[b] = pltpu.async_copy(oh, shared.at[rowb], sems[sl],
                                      add=True)

        for b in (NBATCH - 2, NBATCH - 1):
            cps[b].wait()
            sl = b % 2
            oh, colb = ohs[sl], colbs[sl]

            def clean2(i, carry2):
                col = colb[pl.ds(i * 16, 16)]
                rowi = i * 16 + iota16
                plsc.store_scatter(oh, [rowi, col], zero16)
                return carry2

            lax.fori_loop(0, BATCH // 16, clean2, 0, unroll=5)
        return carry

    lax.fori_loop(0, nch, chunk_body, 0)

    plsc.subcore_barrier()
    pltpu.sync_copy(shared.at[pl.ds(zrow0, ROWS_PER_TILE)],
                    s_hbm.at[c, pl.ds(zrow0, ROWS_PER_TILE)])


def _tc_fuse(s0t_ref, s1t_ref, x_ref, tin_ref, tout_ref, weff_ref, bias_ref,
             gamma_ref, beta_ref, o_ref):
    dn = (((0,), (0,)), ((), ()))
    a0 = lax.dot_general(s0t_ref[...], tin_ref[...], dn,
                         preferred_element_type=jnp.float32)
    a1 = lax.dot_general(s1t_ref[...], tout_ref[...], dn,
                         preferred_element_type=jnp.float32)
    hb = (a0[:N_NODES] + a1[:N_NODES]) / 3.0
    hb = hb + jnp.dot(x_ref[...], weff_ref[...],
                      preferred_element_type=jnp.float32) / 3.0
    hb = hb + bias_ref[...]
    n = jnp.float32(N_NODES)
    mean = jnp.sum(hb, axis=0, keepdims=True) / n
    var = jnp.sum(hb * hb, axis=0, keepdims=True) / n - mean * mean
    o_ref[...] = ((hb - mean) / jnp.sqrt(var + 1e-5)
                  * gamma_ref[...] + beta_ref[...])


def kernel(x, rel_repr, edge_index, edge_type, edge_norm, in_w, out_w,
           loop_w, w_rel, loop_rel, bias, bn_gamma, bn_beta):
    n_edges = edge_type.shape[0]
    half = n_edges // 2
    dst2 = edge_index[1].reshape(n_edges // D, D)
    typ2 = edge_type.reshape(n_edges // D, D)

    r = loop_rel[0]
    rr = jnp.concatenate([r, r])
    circ = jnp.stack([lax.slice(rr, (j,), (j + D,)) for j in range(D)])

    t_in, t_out, weff, out2, flat2 = pl.pallas_call(
        _tc_prep,
        out_shape=[
            jax.ShapeDtypeStruct((NREL, D), jnp.float32),
            jax.ShapeDtypeStruct((NREL, D), jnp.float32),
            jax.ShapeDtypeStruct((D, D), jnp.float32),
            jax.ShapeDtypeStruct((NREL, D), jnp.float32),
            jax.ShapeDtypeStruct((n_edges // D, D), jnp.int32),
        ],
    )(rel_repr, in_w, out_w, w_rel, circ, loop_w, dst2, typ2)

    flat = flat2.reshape(n_edges)
    zeros_stripe = jnp.zeros((ROWS_PER_TILE, 16), jnp.float32)

    mesh = plsc.VectorSubcoreMesh(core_axis_name="c", subcore_axis_name="s",
                                  num_cores=NC, num_subcores=NS)
    s_out = pl.kernel(
        functools.partial(_sc_hist, half=half),
        out_type=jax.ShapeDtypeStruct((NC, SHARED_ROWS, 16), jnp.float32),
        mesh=mesh,
        compiler_params=pltpu.CompilerParams(needs_layout_passes=False,
                                             use_tc_tiling_on_sc=False),
        scratch_types=[
            pltpu.VMEM_SHARED((SHARED_ROWS, 16), jnp.float32),
            pltpu.VMEM((BATCH, 16), jnp.float32),
            pltpu.VMEM((BATCH, 16), jnp.float32),
            pltpu.VMEM((CH,), jnp.int32),
            pltpu.VMEM((CH,), jnp.float32),
            pltpu.VMEM((BATCH,), jnp.int32),
            pltpu.VMEM((BATCH,), jnp.int32),
            pltpu.VMEM((BATCH,), jnp.int32),
            pltpu.VMEM((BATCH,), jnp.int32),
            pltpu.SemaphoreType.DMA,
            pltpu.SemaphoreType.DMA,
        ],
    )(flat, edge_norm, zeros_stripe)

    s_t = s_out.reshape(NC, NREL, DSTP)

    h = pl.pallas_call(
        _tc_fuse,
        out_shape=jax.ShapeDtypeStruct((N_NODES, D), jnp.float32),
    )(s_t[0], s_t[1], x, t_in, t_out, weff, bias.reshape(1, D),
      bn_gamma.reshape(1, D), bn_beta.reshape(1, D))

    return (h, out2)


# speedup vs baseline: 21.1403x; 2.5687x over previous
import functools

import jax
import jax.numpy as jnp
from jax import lax
from jax.experimental import pallas as pl
from jax.experimental.pallas import tpu as pltpu
from jax.experimental.pallas import tpu_sc as plsc

N_NODES = 10000
D = 128
NREL = 200
DSTP = 10112
NLANE_ROWS = NREL * DSTP // D

NC, NS = 2, 16
SC_WORDS = NREL * DSTP
SHARED_ROWS = SC_WORDS // 16
ROWS_PER_TILE = SHARED_ROWS // NS

CH = 400
BATCH = 80
NBATCH = CH // BATCH


def _tc_prep(rel_ref, in_w_ref, out_w_ref, w_rel_ref, circ_ref, loop_w_ref,
             tin_ref, tout_ref, weff_ref, out2_ref):
    rel = rel_ref[...]
    tin_ref[...] = jax.nn.softmax(
        jnp.dot(rel, in_w_ref[...], preferred_element_type=jnp.float32),
        axis=-1)
    tout_ref[...] = jax.nn.softmax(
        jnp.dot(rel, out_w_ref[...], preferred_element_type=jnp.float32),
        axis=-1)
    weff_ref[...] = jnp.dot(circ_ref[...], loop_w_ref[...],
                            preferred_element_type=jnp.float32)
    out2_ref[...] = jnp.dot(rel, w_rel_ref[...],
                            preferred_element_type=jnp.float32)


def _sc_hist(ei_hbm, typ_hbm, norm_hbm, zero_hbm, s_hbm, shared, oh0, oh1,
             dstbuf, typbuf, normbuf, colb0, colb1, rowb0, rowb1, sem0, sem1,
             *, half):
    c = lax.axis_index("c")
    s = lax.axis_index("s")
    edges_per_tile = half // NS
    nch = edges_per_tile // CH
    base_e = c * half + s * edges_per_tile

    zero16 = jnp.zeros((16,), jnp.float32)
    iota16 = lax.iota(jnp.int32, 16)

    ohs = (oh0, oh1)
    colbs = (colb0, colb1)
    rowbs = (rowb0, rowb1)
    sems = (sem0, sem1)

    zrow0 = s * ROWS_PER_TILE

    def zb(i, carry):
        oh0[i, :] = zero16
        oh1[i, :] = zero16
        return carry

    lax.fori_loop(0, BATCH, zb, 0, unroll=8)

    pltpu.sync_copy(zero_hbm, shared.at[pl.ds(zrow0, ROWS_PER_TILE)])
    plsc.subcore_barrier()

    def chunk_body(g, carry):
        eb = base_e + g * CH
        pltpu.sync_copy(ei_hbm.at[1, pl.ds(eb, CH)], dstbuf)
        pltpu.sync_copy(typ_hbm.at[pl.ds(eb, CH)], typbuf)
        pltpu.sync_copy(norm_hbm.at[pl.ds(eb, CH)], normbuf)

        cps = [None] * NBATCH
        for b in range(NBATCH):
            sl = b % 2
            oh, colb, rowb = ohs[sl], colbs[sl], rowbs[sl]
            if b >= 2:
                cps[b - 2].wait()

                def clean(i, carry2):
                    col = colb[pl.ds(i * 16, 16)]
                    rowi = i * 16 + iota16
                    plsc.store_scatter(oh, [rowi, col], zero16)
                    return carry2

                lax.fori_loop(0, BATCH // 16, clean, 0, unroll=5)

            def build(i, carry2):
                dst = dstbuf[pl.ds(b * BATCH + i * 16, 16)]
                typ = typbuf[pl.ds(b * BATCH + i * 16, 16)]
                nm = normbuf[pl.ds(b * BATCH + i * 16, 16)]
                fl = typ * DSTP + dst
                col = lax.bitwise_and(fl, 15)
                row = lax.shift_right_logical(fl, 4)
                rowi = i * 16 + iota16
                plsc.store_scatter(oh, [rowi, col], nm)
                colb[pl.ds(i * 16, 16)] = col
                rowb[pl.ds(i * 16, 16)] = row
                return carry2

            lax.fori_loop(0, BATCH // 16, build, 0, unroll=5)
            cps[b] = pltpu.async_copy(oh, shared.at[rowb], sems[sl],
                                      add=True)

        for b in (NBATCH - 2, NBATCH - 1):
            cps[b].wait()
            sl = b % 2
            oh, colb = ohs[sl], colbs[sl]

            def clean2(i, carry2):
                col = colb[pl.ds(i * 16, 16)]
                rowi = i * 16 + iota16
                plsc.store_scatter(oh, [rowi, col], zero16)
                return carry2

            lax.fori_loop(0, BATCH // 16, clean2, 0, unroll=5)
        return carry

    lax.fori_loop(0, nch, chunk_body, 0)

    plsc.subcore_barrier()

    pltpu.sync_copy(shared.at[pl.ds(zrow0, ROWS_PER_TILE)],
                    s_hbm.at[c, pl.ds(zrow0, ROWS_PER_TILE)])


def _tc_fuse(s_ref, x_ref, tin_ref, tout_ref, weff_ref, bias_ref,
             gamma_ref, beta_ref, o_ref):
    dn = (((0,), (0,)), ((), ()))
    s0 = jnp.reshape(s_ref[0], (NREL, DSTP))
    s1 = jnp.reshape(s_ref[1], (NREL, DSTP))
    a0 = lax.dot_general(s0, tin_ref[...], dn,
                         preferred_element_type=jnp.float32)
    a1 = lax.dot_general(s1, tout_ref[...], dn,
                         preferred_element_type=jnp.float32)
    hb = (a0[:N_NODES] + a1[:N_NODES]) / 3.0
    hb = hb + jnp.dot(x_ref[...], weff_ref[...],
                      preferred_element_type=jnp.float32) / 3.0
    hb = hb + bias_ref[...]
    n = jnp.float32(N_NODES)
    mean = jnp.sum(hb, axis=0, keepdims=True) / n
    var = jnp.sum(hb * hb, axis=0, keepdims=True) / n - mean * mean
    o_ref[...] = ((hb - mean) / jnp.sqrt(var + 1e-5)
                  * gamma_ref[...] + beta_ref[...])


def kernel(x, rel_repr, edge_index, edge_type, edge_norm, in_w, out_w,
           loop_w, w_rel, loop_rel, bias, bn_gamma, bn_beta):
    n_edges = edge_type.shape[0]
    half = n_edges // 2

    r = loop_rel[0]
    rr = jnp.concatenate([r, r])
    circ = jnp.stack([lax.slice(rr, (j,), (j + D,)) for j in range(D)])

    t_in, t_out, weff, out2 = pl.pallas_call(
        _tc_prep,
        out_shape=[
            jax.ShapeDtypeStruct((NREL, D), jnp.float32),
            jax.ShapeDtypeStruct((NREL, D), jnp.float32),
            jax.ShapeDtypeStruct((D, D), jnp.float32),
            jax.ShapeDtypeStruct((NREL, D), jnp.float32),
        ],
    )(rel_repr, in_w, out_w, w_rel, circ, loop_w)

    zeros_stripe = jnp.zeros((ROWS_PER_TILE, 16), jnp.float32)

    mesh = plsc.VectorSubcoreMesh(core_axis_name="c", subcore_axis_name="s",
                                  num_cores=NC, num_subcores=NS)
    s_out = pl.kernel(
        functools.partial(_sc_hist, half=half),
        out_type=jax.ShapeDtypeStruct((NC, SHARED_ROWS, 16), jnp.float32),
        mesh=mesh,
        compiler_params=pltpu.CompilerParams(needs_layout_passes=False,
                                             use_tc_tiling_on_sc=False),
        scratch_types=[
            pltpu.VMEM_SHARED((SHARED_ROWS, 16), jnp.float32),
            pltpu.VMEM((BATCH, 16), jnp.float32),
            pltpu.VMEM((BATCH, 16), jnp.float32),
            pltpu.VMEM((CH,), jnp.int32),
            pltpu.VMEM((CH,), jnp.int32),
            pltpu.VMEM((CH,), jnp.float32),
            pltpu.VMEM((BATCH,), jnp.int32),
            pltpu.VMEM((BATCH,), jnp.int32),
            pltpu.VMEM((BATCH,), jnp.int32),
            pltpu.VMEM((BATCH,), jnp.int32),
            pltpu.SemaphoreType.DMA,
            pltpu.SemaphoreType.DMA,
        ],
    )(edge_index, edge_type, edge_norm, zeros_stripe)

    s128 = s_out.reshape(NC, NLANE_ROWS, 128)

    h = pl.pallas_call(
        _tc_fuse,
        out_shape=jax.ShapeDtypeStruct((N_NODES, D), jnp.float32),
    )(s128, x, t_in, t_out, weff, bias.reshape(1, D),
      bn_gamma.reshape(1, D), bn_beta.reshape(1, D))

    return (h, out2)


# speedup vs baseline: 24.2901x; 1.1490x over previous
import functools

import jax
import jax.numpy as jnp
from jax import lax
from jax.experimental import pallas as pl
from jax.experimental.pallas import tpu as pltpu
from jax.experimental.pallas import tpu_sc as plsc

N_NODES = 10000
D = 128
NREL = 200
DSTP = 10112
NLANE_ROWS = NREL * DSTP // D

NC, NS = 2, 16
SC_WORDS = NREL * DSTP
SHARED_ROWS = SC_WORDS // 16
ROWS_PER_TILE = SHARED_ROWS // NS

CH = 400
BATCH = 80
NBATCH = CH // BATCH


def _tc_prep(rel_ref, in_w_ref, out_w_ref, w_rel_ref, circ_ref, loop_w_ref,
             tin_ref, tout_ref, weff_ref, out2_ref):
    rel = rel_ref[...]
    tin_ref[...] = jax.nn.softmax(
        jnp.dot(rel, in_w_ref[...], preferred_element_type=jnp.float32),
        axis=-1)
    tout_ref[...] = jax.nn.softmax(
        jnp.dot(rel, out_w_ref[...], preferred_element_type=jnp.float32),
        axis=-1)
    weff_ref[...] = jnp.dot(circ_ref[...], loop_w_ref[...],
                            preferred_element_type=jnp.float32)
    out2_ref[...] = jnp.dot(rel, w_rel_ref[...],
                            preferred_element_type=jnp.float32)


def _sc_hist(ei_hbm, typ_hbm, norm_hbm, zero_hbm, s_hbm, shared, oh0, oh1,
             dstbuf, typbuf, normbuf, colb0, colb1, rowb0, rowb1, sem0, sem1,
             stg_sem, *, half):
    c = lax.axis_index("c")
    s = lax.axis_index("s")
    edges_per_tile = half // NS
    nch = edges_per_tile // CH
    base_e = c * half + s * edges_per_tile

    zero16 = jnp.zeros((16,), jnp.float32)
    iota16 = lax.iota(jnp.int32, 16)

    ohs = (oh0, oh1)
    colbs = (colb0, colb1)
    rowbs = (rowb0, rowb1)
    sems = (sem0, sem1)

    zrow0 = s * ROWS_PER_TILE

    def zb(i, carry):
        oh0[i, :] = zero16
        oh1[i, :] = zero16
        return carry

    lax.fori_loop(0, BATCH, zb, 0, unroll=8)

    pltpu.sync_copy(zero_hbm, shared.at[pl.ds(zrow0, ROWS_PER_TILE)])
    plsc.subcore_barrier()

    def make_clean(oh, colb):
        def clean(i, carry2):
            col = colb[pl.ds(i * 16, 16)]
            rowi = i * 16 + iota16
            plsc.store_scatter(oh, [rowi, col], zero16)
            return carry2

        return clean

    def make_build(oh, colb, rowb, b):
        def build(i, carry2):
            dst = dstbuf[pl.ds(b * BATCH + i * 16, 16)]
            typ = typbuf[pl.ds(b * BATCH + i * 16, 16)]
            nm = normbuf[pl.ds(b * BATCH + i * 16, 16)]
            fl = typ * DSTP + dst
            col = lax.bitwise_and(fl, 15)
            row = lax.shift_right_logical(fl, 4)
            rowi = i * 16 + iota16
            plsc.store_scatter(oh, [rowi, col], nm)
            colb[pl.ds(i * 16, 16)] = col
            rowb[pl.ds(i * 16, 16)] = row
            return carry2

        return build

    def stage_async(g):
        eb = base_e + g * CH
        return [
            pltpu.async_copy(ei_hbm.at[1, pl.ds(eb, CH)], dstbuf, stg_sem),
            pltpu.async_copy(typ_hbm.at[pl.ds(eb, CH)], typbuf, stg_sem),
            pltpu.async_copy(norm_hbm.at[pl.ds(eb, CH)], normbuf, stg_sem),
        ]

    outstanding = [None, None]
    for cp in stage_async(0):
        cp.wait()
    bi = 0
    for g in range(nch):
        if g > 0:
            for cp in stg_cps:
                cp.wait()
        for b in range(NBATCH):
            sl = bi % 2
            oh, colb, rowb = ohs[sl], colbs[sl], rowbs[sl]
            if outstanding[sl] is not None:
                outstanding[sl].wait()
                lax.fori_loop(0, BATCH // 16, make_clean(oh, colb), 0,
                              unroll=5)
            lax.fori_loop(0, BATCH // 16, make_build(oh, colb, rowb, b), 0,
                          unroll=5)
            outstanding[sl] = pltpu.async_copy(oh, shared.at[rowb],
                                               sems[sl], add=True)
            bi += 1
            if b == NBATCH - 1 and g + 1 < nch:
                stg_cps = stage_async(g + 1)
    for sl in (0, 1):
        if outstanding[sl] is not None:
            outstanding[sl].wait()

    plsc.subcore_barrier()

    pltpu.sync_copy(shared.at[pl.ds(zrow0, ROWS_PER_TILE)],
                    s_hbm.at[c, pl.ds(zrow0, ROWS_PER_TILE)])


def _tc_fuse(s_ref, x_ref, tin_ref, tout_ref, weff_ref, bias_ref,
             gamma_ref, beta_ref, o_ref):
    dn = (((0,), (0,)), ((), ()))
    s0 = jnp.reshape(s_ref[0], (NREL, DSTP))
    s1 = jnp.reshape(s_ref[1], (NREL, DSTP))
    a0 = lax.dot_general(s0, tin_ref[...], dn,
                         preferred_element_type=jnp.float32)
    a1 = lax.dot_general(s1, tout_ref[...], dn,
                         preferred_element_type=jnp.float32)
    hb = (a0[:N_NODES] + a1[:N_NODES]) / 3.0
    hb = hb + jnp.dot(x_ref[...], weff_ref[...],
                      preferred_element_type=jnp.float32) / 3.0
    hb = hb + bias_ref[...]
    n = jnp.float32(N_NODES)
    mean = jnp.sum(hb, axis=0, keepdims=True) / n
    var = jnp.sum(hb * hb, axis=0, keepdims=True) / n - mean * mean
    o_ref[...] = ((hb - mean) / jnp.sqrt(var + 1e-5)
                  * gamma_ref[...] + beta_ref[...])


def kernel(x, rel_repr, edge_index, edge_type, edge_norm, in_w, out_w,
           loop_w, w_rel, loop_rel, bias, bn_gamma, bn_beta):
    n_edges = edge_type.shape[0]
    half = n_edges // 2

    r = loop_rel[0]
    v = jnp.tile(r, D + 2)
    circ = v[:D * (D + 1)].reshape(D, D + 1)[:, :D]

    zeros_stripe = jnp.zeros((ROWS_PER_TILE, 16), jnp.float32)

    mesh = plsc.VectorSubcoreMesh(core_axis_name="c", subcore_axis_name="s",
                                  num_cores=NC, num_subcores=NS)
    s_out = pl.kernel(
        functools.partial(_sc_hist, half=half),
        out_type=jax.ShapeDtypeStruct((NC, SHARED_ROWS, 16), jnp.float32),
        mesh=mesh,
        compiler_params=pltpu.CompilerParams(needs_layout_passes=False,
                                             use_tc_tiling_on_sc=False),
        scratch_types=[
            pltpu.VMEM_SHARED((SHARED_ROWS, 16), jnp.float32),
            pltpu.VMEM((BATCH, 16), jnp.float32),
            pltpu.VMEM((BATCH, 16), jnp.float32),
            pltpu.VMEM((CH,), jnp.int32),
            pltpu.VMEM((CH,), jnp.int32),
            pltpu.VMEM((CH,), jnp.float32),
            pltpu.VMEM((BATCH,), jnp.int32),
            pltpu.VMEM((BATCH,), jnp.int32),
            pltpu.VMEM((BATCH,), jnp.int32),
            pltpu.VMEM((BATCH,), jnp.int32),
            pltpu.SemaphoreType.DMA,
            pltpu.SemaphoreType.DMA,
            pltpu.SemaphoreType.DMA,
        ],
    )(edge_index, edge_type, edge_norm, zeros_stripe)

    t_in, t_out, weff, out2 = pl.pallas_call(
        _tc_prep,
        out_shape=[
            jax.ShapeDtypeStruct((NREL, D), jnp.float32),
            jax.ShapeDtypeStruct((NREL, D), jnp.float32),
            jax.ShapeDtypeStruct((D, D), jnp.float32),
            jax.ShapeDtypeStruct((NREL, D), jnp.float32),
        ],
    )(rel_repr, in_w, out_w, w_rel, circ, loop_w)

    s128 = s_out.reshape(NC, NLANE_ROWS, 128)

    h = pl.pallas_call(
        _tc_fuse,
        out_shape=jax.ShapeDtypeStruct((N_NODES, D), jnp.float32),
    )(s128, x, t_in, t_out, weff, bias.reshape(1, D),
      bn_gamma.reshape(1, D), bn_beta.reshape(1, D))

    return (h, out2)
